# baseline (speedup 1.0000x reference)
import functools

import jax
import jax.numpy as jnp
from jax import lax
from jax.experimental import pallas as pl
from jax.experimental.pallas import tpu as pltpu
from jax.experimental.pallas import tpu_sc as plsc


def _sc_gather_t(table_t_flat, idx, V, D, B):
    info = plsc.get_sparse_core_info()
    NC, NS, L = info.num_cores, info.num_subcores, info.num_lanes
    NW = NC * NS
    w_per_d = NW // D
    chunk = B // w_per_d
    n_idx_dma = chunk // 128
    mesh = plsc.VectorSubcoreMesh(core_axis_name="c", subcore_axis_name="s")

    @functools.partial(
        pl.kernel,
        mesh=mesh,
        compiler_params=pltpu.CompilerParams(use_tc_tiling_on_sc=False),
        out_type=jax.ShapeDtypeStruct((D, B), jnp.float32),
        scratch_types=[
            pltpu.VMEM((chunk,), jnp.int32),
            pltpu.VMEM((chunk,), jnp.int32),
            pltpu.VMEM((chunk,), jnp.float32),
            pltpu.SemaphoreType.DMA,
        ],
    )
    def gather_kernel(tflat_hbm, idx_hbm, out_hbm, idx_v, addr_v, dst_v, sem):
        wid = lax.axis_index("s") * NC + lax.axis_index("c")
        d = wid // w_per_d
        base = (wid % w_per_d) * chunk
        pltpu.sync_copy(idx_hbm.at[pl.ds(base, chunk)], idx_v)
        off = d * V
        for c in range(chunk // L):
            addr_v[pl.ds(c * L, L)] = idx_v[pl.ds(c * L, L)] + off
        copies = [
            pltpu.async_copy(
                tflat_hbm.at[addr_v.at[pl.ds(k * 128, 128)]],
                dst_v.at[pl.ds(k * 128, 128)],
                sem,
            )
            for k in range(n_idx_dma)
        ]
        for cp in copies:
            cp.wait()
        pltpu.sync_copy(dst_v, out_hbm.at[d, pl.ds(base, chunk)])

    return gather_kernel(table_t_flat, idx)


def _tc_project(eT, lin_w, lin_b, block_v):
    D, B = eT.shape
    V = lin_w.shape[0]
    nv = pl.cdiv(V, block_v)
    wT = lin_w.T
    lin_b2 = lin_b.reshape(1, V)

    def body(w_ref, b_ref, e_ref, o_ref):
        wa = jnp.concatenate([w_ref[...], b_ref[...]], axis=0)
        ea = jnp.concatenate([e_ref[...], jnp.ones((1, B), jnp.float32)], axis=0)
        o_ref[...] = lax.dot_general(
            wa,
            ea,
            dimension_numbers=(((0,), (0,)), ((), ())),
            preferred_element_type=jnp.float32,
        )

    outT = pl.pallas_call(
        body,
        grid=(nv,),
        in_specs=[
            pl.BlockSpec((D, block_v), lambda j: (0, j)),
            pl.BlockSpec((1, block_v), lambda j: (0, j)),
            pl.BlockSpec((D, B), lambda j: (0, 0)),
        ],
        out_specs=pl.BlockSpec((block_v, B), lambda j: (j, 0)),
        out_shape=jax.ShapeDtypeStruct((V, B), jnp.float32),
    )(wT, lin_b2, eT)
    return outT.T


def kernel(inputs, emb_table, lin_w, lin_b):
    idx = inputs.astype(jnp.int32)
    V, D = emb_table.shape
    (B,) = idx.shape
    table_t_flat = emb_table.T.reshape(-1)
    eT = _sc_gather_t(table_t_flat, idx, V, D, B)
    return _tc_project(eT, lin_w, lin_b, block_v=2048)


# speedup vs baseline: 1.0013x; 1.0013x over previous
import functools

import jax
import jax.numpy as jnp
from jax import lax
from jax.experimental import pallas as pl
from jax.experimental.pallas import tpu as pltpu
from jax.experimental.pallas import tpu_sc as plsc


def _sc_gather_t(table_t_flat, idx, V, D, B):
    info = plsc.get_sparse_core_info()
    NC, NS, L = info.num_cores, info.num_subcores, info.num_lanes
    NW = NC * NS
    w_per_d = NW // D
    chunk = B // w_per_d
    n_idx_dma = chunk // 128
    mesh = plsc.VectorSubcoreMesh(core_axis_name="c", subcore_axis_name="s")

    @functools.partial(
        pl.kernel,
        mesh=mesh,
        compiler_params=pltpu.CompilerParams(use_tc_tiling_on_sc=False),
        out_type=jax.ShapeDtypeStruct((D, B), jnp.float32),
        scratch_types=[
            pltpu.VMEM((chunk,), jnp.int32),
            pltpu.VMEM((chunk,), jnp.int32),
            pltpu.VMEM((chunk,), jnp.float32),
            pltpu.SemaphoreType.DMA,
        ],
    )
    def gather_kernel(tflat_hbm, idx_hbm, out_hbm, idx_v, addr_v, dst_v, sem):
        wid = lax.axis_index("s") * NC + lax.axis_index("c")
        d = wid // w_per_d
        base = (wid % w_per_d) * chunk
        pltpu.sync_copy(idx_hbm.at[pl.ds(base, chunk)], idx_v)
        off = d * V
        for c in range(chunk // L):
            addr_v[pl.ds(c * L, L)] = idx_v[pl.ds(c * L, L)] + off
        copies = [
            pltpu.async_copy(
                tflat_hbm.at[addr_v.at[pl.ds(k * 128, 128)]],
                dst_v.at[pl.ds(k * 128, 128)],
                sem,
            )
            for k in range(n_idx_dma)
        ]
        for cp in copies:
            cp.wait()
        pltpu.sync_copy(dst_v, out_hbm.at[d, pl.ds(base, chunk)])

    return gather_kernel(table_t_flat, idx)


def _sc_gather_rowstage(table_t, idx, V, D, B):
    info = plsc.get_sparse_core_info()
    NC, NS, L = info.num_cores, info.num_subcores, info.num_lanes
    NW = NC * NS
    w_per_d = NW // D
    chunk = B // w_per_d
    mesh = plsc.VectorSubcoreMesh(core_axis_name="c", subcore_axis_name="s")

    @functools.partial(
        pl.kernel,
        mesh=mesh,
        compiler_params=pltpu.CompilerParams(use_tc_tiling_on_sc=False),
        out_type=jax.ShapeDtypeStruct((D, B), jnp.float32),
        scratch_types=[
            pltpu.VMEM((V,), jnp.float32),
            pltpu.VMEM((chunk,), jnp.int32),
            pltpu.VMEM((chunk,), jnp.float32),
            pltpu.SemaphoreType.DMA,
        ],
    )
    def gather_kernel(table_hbm, idx_hbm, out_hbm, row_v, idx_v, dst_v, sem):
        wid = lax.axis_index("s") * NC + lax.axis_index("c")
        d = wid // w_per_d
        base = (wid % w_per_d) * chunk
        pltpu.sync_copy(idx_hbm.at[pl.ds(base, chunk)], idx_v)
        copies = [
            pltpu.async_copy(
                table_hbm.at[d].at[idx_v.at[pl.ds(k * 128, 128)]],
                dst_v.at[pl.ds(k * 128, 128)],
                sem,
            )
            for k in range(chunk // 128)
        ]
        for cp in copies:
            cp.wait()
        pltpu.sync_copy(dst_v, out_hbm.at[d, pl.ds(base, chunk)])

    return gather_kernel(table_t, idx)


def _tc_project(eT, lin_w, lin_b, block_v):
    D, B = eT.shape
    V = lin_w.shape[0]
    nv = pl.cdiv(V, block_v)
    wT = lin_w.T
    lin_b2 = lin_b.reshape(1, V)

    def body(w_ref, b_ref, e_ref, o_ref):
        wa = jnp.concatenate([w_ref[...], b_ref[...]], axis=0)
        ea = jnp.concatenate([e_ref[...], jnp.ones((1, B), jnp.float32)], axis=0)
        o_ref[...] = lax.dot_general(
            wa,
            ea,
            dimension_numbers=(((0,), (0,)), ((), ())),
            preferred_element_type=jnp.float32,
        )

    outT = pl.pallas_call(
        body,
        grid=(nv,),
        in_specs=[
            pl.BlockSpec((D, block_v), lambda j: (0, j)),
            pl.BlockSpec((1, block_v), lambda j: (0, j)),
            pl.BlockSpec((D, B), lambda j: (0, 0)),
        ],
        out_specs=pl.BlockSpec((block_v, B), lambda j: (j, 0)),
        out_shape=jax.ShapeDtypeStruct((V, B), jnp.float32),
    )(wT, lin_b2, eT)
    return outT.T


def kernel(inputs, emb_table, lin_w, lin_b):
    idx = inputs.astype(jnp.int32)
    V, D = emb_table.shape
    (B,) = idx.shape
    eT = _sc_gather_rowstage(emb_table.T, idx, V, D, B)
    return _tc_project(eT, lin_w, lin_b, block_v=2048)


# speedup vs baseline: 1.0084x; 1.0072x over previous
import functools

import jax
import jax.numpy as jnp
from jax import lax
from jax.experimental import pallas as pl
from jax.experimental.pallas import tpu as pltpu
from jax.experimental.pallas import tpu_sc as plsc


def _sc_gather_t(table_t_flat, idx, V, D, B):
    info = plsc.get_sparse_core_info()
    NC, NS, L = info.num_cores, info.num_subcores, info.num_lanes
    NW = NC * NS
    w_per_d = NW // D
    chunk = B // w_per_d
    n_idx_dma = chunk // 128
    mesh = plsc.VectorSubcoreMesh(core_axis_name="c", subcore_axis_name="s")

    @functools.partial(
        pl.kernel,
        mesh=mesh,
        compiler_params=pltpu.CompilerParams(use_tc_tiling_on_sc=False),
        out_type=jax.ShapeDtypeStruct((D, B), jnp.float32),
        scratch_types=[
            pltpu.VMEM((chunk,), jnp.int32),
            pltpu.VMEM((chunk,), jnp.int32),
            pltpu.VMEM((chunk,), jnp.float32),
            pltpu.SemaphoreType.DMA,
        ],
    )
    def gather_kernel(tflat_hbm, idx_hbm, out_hbm, idx_v, addr_v, dst_v, sem):
        wid = lax.axis_index("s") * NC + lax.axis_index("c")
        d = wid // w_per_d
        base = (wid % w_per_d) * chunk
        pltpu.sync_copy(idx_hbm.at[pl.ds(base, chunk)], idx_v)
        off = d * V
        for c in range(chunk // L):
            addr_v[pl.ds(c * L, L)] = idx_v[pl.ds(c * L, L)] + off
        copies = [
            pltpu.async_copy(
                tflat_hbm.at[addr_v.at[pl.ds(k * 128, 128)]],
                dst_v.at[pl.ds(k * 128, 128)],
                sem,
            )
            for k in range(n_idx_dma)
        ]
        for cp in copies:
            cp.wait()
        pltpu.sync_copy(dst_v, out_hbm.at[d, pl.ds(base, chunk)])

    return gather_kernel(table_t_flat, idx)


def _sc_gather_rowstage(table_t, idx, V, D, B):
    info = plsc.get_sparse_core_info()
    NC, NS, L = info.num_cores, info.num_subcores, info.num_lanes
    NC = 1
    NW = NC * NS
    w_per_d = max(1, NW // D)
    chunk = B // w_per_d
    mesh = plsc.VectorSubcoreMesh(core_axis_name="c", subcore_axis_name="s", num_cores=1)

    @functools.partial(
        pl.kernel,
        mesh=mesh,
        compiler_params=pltpu.CompilerParams(use_tc_tiling_on_sc=False),
        out_type=jax.ShapeDtypeStruct((D, B), jnp.float32),
        scratch_types=[
            pltpu.VMEM((V,), jnp.float32),
            pltpu.VMEM((chunk,), jnp.int32),
            pltpu.VMEM((chunk,), jnp.float32),
            pltpu.SemaphoreType.DMA,
        ],
    )
    def gather_kernel(table_hbm, idx_hbm, out_hbm, row_v, idx_v, dst_v, sem):
        wid = lax.axis_index("s") * NC + lax.axis_index("c")
        d = wid // w_per_d
        base = (wid % w_per_d) * chunk
        pltpu.sync_copy(idx_hbm.at[pl.ds(base, chunk)], idx_v)
        copies = [
            pltpu.async_copy(
                table_hbm.at[d].at[idx_v.at[pl.ds(k * 128, 128)]],
                dst_v.at[pl.ds(k * 128, 128)],
                sem,
            )
            for k in range(chunk // 128)
        ]
        for cp in copies:
            cp.wait()
        pltpu.sync_copy(dst_v, out_hbm.at[d, pl.ds(base, chunk)])

    return gather_kernel(table_t, idx)


def _tc_project(eT, lin_w, lin_b, block_v):
    D, B = eT.shape
    V = lin_w.shape[0]
    nv = pl.cdiv(V, block_v)
    wT = lin_w.T
    lin_b2 = lin_b.reshape(1, V)

    def body(w_ref, b_ref, e_ref, o_ref):
        wa = jnp.concatenate([w_ref[...], b_ref[...]], axis=0)
        ea = jnp.concatenate([e_ref[...], jnp.ones((1, B), jnp.float32)], axis=0)
        o_ref[...] = lax.dot_general(
            wa,
            ea,
            dimension_numbers=(((0,), (0,)), ((), ())),
            preferred_element_type=jnp.float32,
        )

    outT = pl.pallas_call(
        body,
        grid=(nv,),
        in_specs=[
            pl.BlockSpec((D, block_v), lambda j: (0, j)),
            pl.BlockSpec((1, block_v), lambda j: (0, j)),
            pl.BlockSpec((D, B), lambda j: (0, 0)),
        ],
        out_specs=pl.BlockSpec((block_v, B), lambda j: (j, 0)),
        out_shape=jax.ShapeDtypeStruct((V, B), jnp.float32),
    )(wT, lin_b2, eT)
    return outT.T


def kernel(inputs, emb_table, lin_w, lin_b):
    idx = inputs.astype(jnp.int32)
    V, D = emb_table.shape
    (B,) = idx.shape
    eT = _sc_gather_rowstage(emb_table.T, idx, V, D, B)
    return _tc_project(eT, lin_w, lin_b, block_v=2048)


# speedup vs baseline: 1.0091x; 1.0006x over previous
import functools

import jax
import jax.numpy as jnp
from jax import lax
from jax.experimental import pallas as pl
from jax.experimental.pallas import tpu as pltpu
from jax.experimental.pallas import tpu_sc as plsc


def _sc_gather_t(table_t_flat, idx, V, D, B):
    info = plsc.get_sparse_core_info()
    NC, NS, L = info.num_cores, info.num_subcores, info.num_lanes
    NW = NC * NS
    w_per_d = NW // D
    chunk = B // w_per_d
    n_idx_dma = chunk // 128
    mesh = plsc.VectorSubcoreMesh(core_axis_name="c", subcore_axis_name="s")

    @functools.partial(
        pl.kernel,
        mesh=mesh,
        compiler_params=pltpu.CompilerParams(use_tc_tiling_on_sc=False),
        out_type=jax.ShapeDtypeStruct((D, B), jnp.float32),
        scratch_types=[
            pltpu.VMEM((chunk,), jnp.int32),
            pltpu.VMEM((chunk,), jnp.int32),
            pltpu.VMEM((chunk,), jnp.float32),
            pltpu.SemaphoreType.DMA,
        ],
    )
    def gather_kernel(tflat_hbm, idx_hbm, out_hbm, idx_v, addr_v, dst_v, sem):
        wid = lax.axis_index("s") * NC + lax.axis_index("c")
        d = wid // w_per_d
        base = (wid % w_per_d) * chunk
        pltpu.sync_copy(idx_hbm.at[pl.ds(base, chunk)], idx_v)
        off = d * V
        for c in range(chunk // L):
            addr_v[pl.ds(c * L, L)] = idx_v[pl.ds(c * L, L)] + off
        copies = [
            pltpu.async_copy(
                tflat_hbm.at[addr_v.at[pl.ds(k * 128, 128)]],
                dst_v.at[pl.ds(k * 128, 128)],
                sem,
            )
            for k in range(n_idx_dma)
        ]
        for cp in copies:
            cp.wait()
        pltpu.sync_copy(dst_v, out_hbm.at[d, pl.ds(base, chunk)])

    return gather_kernel(table_t_flat, idx)


def _sc_gather_rowstage(table_t, idx, V, D, B):
    info = plsc.get_sparse_core_info()
    NC, NS, L = info.num_cores, info.num_subcores, info.num_lanes
    NC = 1
    NW = NC * NS
    w_per_d = max(1, NW // D)
    chunk = B // w_per_d
    mesh = plsc.VectorSubcoreMesh(core_axis_name="c", subcore_axis_name="s", num_cores=1)

    @functools.partial(
        pl.kernel,
        mesh=mesh,
        compiler_params=pltpu.CompilerParams(use_tc_tiling_on_sc=False),
        out_type=jax.ShapeDtypeStruct((D, B), jnp.float32),
        scratch_types=[
            pltpu.VMEM((V,), jnp.float32),
            pltpu.VMEM((chunk,), jnp.int32),
            pltpu.VMEM((chunk,), jnp.float32),
            pltpu.SemaphoreType.DMA,
        ],
    )
    def gather_kernel(table_hbm, idx_hbm, out_hbm, row_v, idx_v, dst_v, sem):
        wid = lax.axis_index("s") * NC + lax.axis_index("c")
        d = wid // w_per_d
        base = (wid % w_per_d) * chunk
        pltpu.sync_copy(idx_hbm.at[pl.ds(base, chunk)], idx_v)
        copies = [
            pltpu.async_copy(
                table_hbm.at[d].at[idx_v.at[pl.ds(k * 128, 128)]],
                dst_v.at[pl.ds(k * 128, 128)],
                sem,
            )
            for k in range(chunk // 128)
        ]
        for cp in copies:
            cp.wait()
        pltpu.sync_copy(dst_v, out_hbm.at[d, pl.ds(base, chunk)])

    return gather_kernel(table_t, idx)


def _tc_project(eT, lin_w, lin_b, block_v):
    D, B = eT.shape
    V = lin_w.shape[0]
    nv = pl.cdiv(V, block_v)
    wT = lin_w.T
    lin_b2 = lin_b.reshape(1, V)

    def body(w_ref, b_ref, e_ref, o_ref):
        wa = jnp.concatenate([w_ref[...], b_ref[...]], axis=0)
        ea = jnp.concatenate([e_ref[...], jnp.ones((1, B), jnp.float32)], axis=0)
        o_ref[...] = lax.dot_general(
            wa,
            ea,
            dimension_numbers=(((0,), (0,)), ((), ())),
            preferred_element_type=jnp.float32,
        )

    outT = pl.pallas_call(
        body,
        grid=(nv,),
        in_specs=[
            pl.BlockSpec((D, block_v), lambda j: (0, j)),
            pl.BlockSpec((1, block_v), lambda j: (0, j)),
            pl.BlockSpec((D, B), lambda j: (0, 0)),
        ],
        out_specs=pl.BlockSpec((block_v, B), lambda j: (j, 0)),
        out_shape=jax.ShapeDtypeStruct((V, B), jnp.float32),
    )(wT, lin_b2, eT)
    return outT.T


def kernel(inputs, emb_table, lin_w, lin_b):
    idx = inputs.astype(jnp.int32)
    V, D = emb_table.shape
    (B,) = idx.shape
    eT = _sc_gather_rowstage(emb_table.T, idx, V, D, B)
    return _tc_project(eT, lin_w, lin_b, block_v=2560)


# speedup vs baseline: 1.0175x; 1.0083x over previous
import functools

import jax
import jax.numpy as jnp
from jax import lax
from jax.experimental import pallas as pl
from jax.experimental.pallas import tpu as pltpu
from jax.experimental.pallas import tpu_sc as plsc


def _sc_gather_t(table_t_flat, idx, V, D, B):
    info = plsc.get_sparse_core_info()
    NC, NS, L = info.num_cores, info.num_subcores, info.num_lanes
    NW = NC * NS
    w_per_d = NW // D
    chunk = B // w_per_d
    n_idx_dma = chunk // 128
    mesh = plsc.VectorSubcoreMesh(core_axis_name="c", subcore_axis_name="s")

    @functools.partial(
        pl.kernel,
        mesh=mesh,
        compiler_params=pltpu.CompilerParams(use_tc_tiling_on_sc=False),
        out_type=jax.ShapeDtypeStruct((D, B), jnp.float32),
        scratch_types=[
            pltpu.VMEM((chunk,), jnp.int32),
            pltpu.VMEM((chunk,), jnp.int32),
            pltpu.VMEM((chunk,), jnp.float32),
            pltpu.SemaphoreType.DMA,
        ],
    )
    def gather_kernel(tflat_hbm, idx_hbm, out_hbm, idx_v, addr_v, dst_v, sem):
        wid = lax.axis_index("s") * NC + lax.axis_index("c")
        d = wid // w_per_d
        base = (wid % w_per_d) * chunk
        pltpu.sync_copy(idx_hbm.at[pl.ds(base, chunk)], idx_v)
        off = d * V
        for c in range(chunk // L):
            addr_v[pl.ds(c * L, L)] = idx_v[pl.ds(c * L, L)] + off
        copies = [
            pltpu.async_copy(
                tflat_hbm.at[addr_v.at[pl.ds(k * 128, 128)]],
                dst_v.at[pl.ds(k * 128, 128)],
                sem,
            )
            for k in range(n_idx_dma)
        ]
        for cp in copies:
            cp.wait()
        pltpu.sync_copy(dst_v, out_hbm.at[d, pl.ds(base, chunk)])

    return gather_kernel(table_t_flat, idx)


def _sc_gather_rowstage(table_t, idx, V, D, B):
    info = plsc.get_sparse_core_info()
    NC, NS, L = info.num_cores, info.num_subcores, info.num_lanes
    NC = 1
    NW = NC * NS
    w_per_d = max(1, NW // D)
    chunk = B // w_per_d
    mesh = plsc.VectorSubcoreMesh(core_axis_name="c", subcore_axis_name="s", num_cores=1)

    @functools.partial(
        pl.kernel,
        mesh=mesh,
        compiler_params=pltpu.CompilerParams(use_tc_tiling_on_sc=False),
        out_type=jax.ShapeDtypeStruct((D, B), jnp.float32),
        scratch_types=[
            pltpu.VMEM((V,), jnp.float32),
            pltpu.VMEM((chunk,), jnp.int32),
            pltpu.VMEM((chunk,), jnp.float32),
            pltpu.SemaphoreType.DMA,
        ],
    )
    def gather_kernel(table_hbm, idx_hbm, out_hbm, row_v, idx_v, dst_v, sem):
        wid = lax.axis_index("s") * NC + lax.axis_index("c")
        d = wid // w_per_d
        base = (wid % w_per_d) * chunk
        pltpu.sync_copy(idx_hbm.at[pl.ds(base, chunk)], idx_v)
        copies = [
            pltpu.async_copy(
                table_hbm.at[d].at[idx_v.at[pl.ds(k * 128, 128)]],
                dst_v.at[pl.ds(k * 128, 128)],
                sem,
            )
            for k in range(chunk // 128)
        ]
        for cp in copies:
            cp.wait()
        pltpu.sync_copy(dst_v, out_hbm.at[d, pl.ds(base, chunk)])

    return gather_kernel(table_t, idx)


def _tc_project(eT, lin_w, lin_b, block_v):
    D, B = eT.shape
    V = lin_w.shape[0]
    nv = pl.cdiv(V, block_v)
    wT = lin_w.T
    lin_b2 = lin_b.reshape(1, V)

    def body(w_ref, b_ref, e_ref, o_ref):
        wa = jnp.concatenate([w_ref[...], b_ref[...]], axis=0)
        ea = jnp.concatenate([e_ref[...], jnp.ones((1, B), jnp.float32)], axis=0)
        o_ref[...] = lax.dot_general(
            wa,
            ea,
            dimension_numbers=(((0,), (0,)), ((), ())),
            preferred_element_type=jnp.float32,
        )

    outT = pl.pallas_call(
        body,
        grid=(nv,),
        in_specs=[
            pl.BlockSpec((D, block_v), lambda j: (0, j)),
            pl.BlockSpec((1, block_v), lambda j: (0, j)),
            pl.BlockSpec((D, B), lambda j: (0, 0)),
        ],
        out_specs=pl.BlockSpec((block_v, B), lambda j: (j, 0)),
        out_shape=jax.ShapeDtypeStruct((V, B), jnp.float32),
    )(wT, lin_b2, eT)
    return outT.T


def kernel(inputs, emb_table, lin_w, lin_b):
    idx = inputs.astype(jnp.int32)
    V, D = emb_table.shape
    (B,) = idx.shape
    eT = _sc_gather_rowstage(emb_table.T, idx, V, D, B)
    return _tc_project(eT, lin_w, lin_b, block_v=1664)
